# baseline (speedup 1.0000x reference)
import functools

import jax
import jax.numpy as jnp
from jax import lax
from jax.experimental import pallas as pl
from jax.experimental.pallas import tpu as pltpu
from jax.experimental.pallas import tpu_sc as plsc

_H = 24
_W = 24
_D2 = 384
_D = 768


def _pos_body(row_hbm, col_hbm, pos_hbm, pos_v):
    wid = lax.axis_index("s") * 2 + lax.axis_index("c")

    @pl.when(wid < _H)
    def _():
        r = wid
        pltpu.sync_copy(col_hbm.at[pl.ds(0, _W)], pos_v.at[:, pl.ds(_D2, _D2)])
        for c in range(_W):
            pltpu.sync_copy(row_hbm.at[r], pos_v.at[c, pl.ds(0, _D2)])
        pltpu.sync_copy(pos_v, pos_hbm.at[r])


@functools.partial(
    pl.kernel,
    out_type=jax.ShapeDtypeStruct((_H, _W, _D), jnp.float32),
    mesh=plsc.VectorSubcoreMesh(core_axis_name="c", subcore_axis_name="s"),
    scratch_types=[
        pltpu.VMEM((_W, _D), jnp.float32),
    ],
)
def _build_pos(row_hbm, col_hbm, pos_hbm, pos_v):
    _pos_body(row_hbm, col_hbm, pos_hbm, pos_v)


def _add_body(x_ref, pos_ref, o_ref):
    o_ref[...] = x_ref[...] + pos_ref[...]


def kernel(x, h, w, row_embed, col_embed):
    B, P, D = x.shape
    pos = _build_pos(row_embed, col_embed).reshape(1, P, D)
    return pl.pallas_call(
        _add_body,
        grid=(B,),
        in_specs=[
            pl.BlockSpec((1, P, D), lambda b: (b, 0, 0)),
            pl.BlockSpec((1, P, D), lambda b: (0, 0, 0)),
        ],
        out_specs=pl.BlockSpec((1, P, D), lambda b: (b, 0, 0)),
        out_shape=jax.ShapeDtypeStruct((B, P, D), jnp.float32),
    )(x, pos)


# speedup vs baseline: 1.8416x; 1.8416x over previous
import functools

import jax
import jax.numpy as jnp
from jax import lax
from jax.experimental import pallas as pl
from jax.experimental.pallas import tpu as pltpu
from jax.experimental.pallas import tpu_sc as plsc

_H = 24
_W = 24
_D2 = 384
_D = 768
_P = _H * _W


def _add_body(row_ref, col_ref, x_ref, o_ref, pos_ref):
    @pl.when(pl.program_id(0) == 0)
    def _():
        re = jnp.broadcast_to(row_ref[...][:, None, :], (_H, _W, _D2))
        ce = jnp.broadcast_to(col_ref[...][None, :, :], (_H, _W, _D2))
        pos_ref[...] = jnp.concatenate([re, ce], axis=-1).reshape(_P, _D)

    o_ref[...] = x_ref[...] + pos_ref[...][None]


def kernel(x, h, w, row_embed, col_embed):
    B, P, D = x.shape
    return pl.pallas_call(
        _add_body,
        grid=(B,),
        in_specs=[
            pl.BlockSpec((_H, _D2), lambda b: (0, 0)),
            pl.BlockSpec((_W, _D2), lambda b: (0, 0)),
            pl.BlockSpec((1, P, D), lambda b: (b, 0, 0)),
        ],
        out_specs=pl.BlockSpec((1, P, D), lambda b: (b, 0, 0)),
        out_shape=jax.ShapeDtypeStruct((B, P, D), jnp.float32),
        scratch_shapes=[pltpu.VMEM((P, D), jnp.float32)],
    )(row_embed, col_embed, x)


# speedup vs baseline: 1.8738x; 1.0175x over previous
import functools

import jax
import jax.numpy as jnp
from jax import lax
from jax.experimental import pallas as pl
from jax.experimental.pallas import tpu as pltpu
from jax.experimental.pallas import tpu_sc as plsc

_H = 24
_W = 24
_D2 = 384
_D = 768
_P = _H * _W


def _add_body(row_ref, col_ref, x_ref, o_ref):
    xb = x_ref[...]
    o_ref[:, :, :, : _D2] = xb[:, :, :, : _D2] + row_ref[...][None, :, None, :]
    o_ref[:, :, :, _D2:] = xb[:, :, :, _D2:] + col_ref[...][None, None, :, :]


def kernel(x, h, w, row_embed, col_embed):
    B, P, D = x.shape
    x4 = x.reshape(B, _H, _W, D)
    out = pl.pallas_call(
        _add_body,
        grid=(B,),
        in_specs=[
            pl.BlockSpec((_H, _D2), lambda b: (0, 0)),
            pl.BlockSpec((_W, _D2), lambda b: (0, 0)),
            pl.BlockSpec((1, _H, _W, D), lambda b: (b, 0, 0, 0)),
        ],
        out_specs=pl.BlockSpec((1, _H, _W, D), lambda b: (b, 0, 0, 0)),
        out_shape=jax.ShapeDtypeStruct((B, _H, _W, D), jnp.float32),
    )(row_embed, col_embed, x4)
    return out.reshape(B, P, D)


# speedup vs baseline: 2.1480x; 1.1464x over previous
import functools

import jax
import jax.numpy as jnp
from jax import lax
from jax.experimental import pallas as pl
from jax.experimental.pallas import tpu as pltpu
from jax.experimental.pallas import tpu_sc as plsc

_H = 24
_W = 24
_D2 = 384
_D = 768
_P = _H * _W


def _add_body(row_ref, col_ref, x_ref, o_ref):
    xb = x_ref[...]
    o_ref[:, :, :, : _D2] = xb[:, :, :, : _D2] + row_ref[...][None, :, None, :]
    o_ref[:, :, :, _D2:] = xb[:, :, :, _D2:] + col_ref[...][None, None, :, :]


def kernel(x, h, w, row_embed, col_embed):
    B, P, D = x.shape
    x4 = x.reshape(B, _H, _W, D)
    out = pl.pallas_call(
        _add_body,
        grid=(B // 2,),
        in_specs=[
            pl.BlockSpec((_H, _D2), lambda b: (0, 0)),
            pl.BlockSpec((_W, _D2), lambda b: (0, 0)),
            pl.BlockSpec((2, _H, _W, D), lambda b: (b, 0, 0, 0)),
        ],
        out_specs=pl.BlockSpec((2, _H, _W, D), lambda b: (b, 0, 0, 0)),
        out_shape=jax.ShapeDtypeStruct((B, _H, _W, D), jnp.float32),
    )(row_embed, col_embed, x4)
    return out.reshape(B, P, D)


# speedup vs baseline: 2.2473x; 1.0462x over previous
import functools

import jax
import jax.numpy as jnp
from jax import lax
from jax.experimental import pallas as pl
from jax.experimental.pallas import tpu as pltpu
from jax.experimental.pallas import tpu_sc as plsc

_H = 24
_W = 24
_D2 = 384
_D = 768
_P = _H * _W


def _add_body(row_ref, col_ref, x_ref, o_ref):
    xb = x_ref[...]
    o_ref[:, :, :, : _D2] = xb[:, :, :, : _D2] + row_ref[...][None, :, None, :]
    o_ref[:, :, :, _D2:] = xb[:, :, :, _D2:] + col_ref[...][None, None, :, :]


def kernel(x, h, w, row_embed, col_embed):
    B, P, D = x.shape
    x4 = x.reshape(B, _H, _W, D)
    out = pl.pallas_call(
        _add_body,
        grid=(B // 4,),
        in_specs=[
            pl.BlockSpec((_H, _D2), lambda b: (0, 0)),
            pl.BlockSpec((_W, _D2), lambda b: (0, 0)),
            pl.BlockSpec((4, _H, _W, D), lambda b: (b, 0, 0, 0)),
        ],
        out_specs=pl.BlockSpec((4, _H, _W, D), lambda b: (b, 0, 0, 0)),
        out_shape=jax.ShapeDtypeStruct((B, _H, _W, D), jnp.float32),
    )(row_embed, col_embed, x4)
    return out.reshape(B, P, D)


# speedup vs baseline: 2.3022x; 1.0244x over previous
import functools

import jax
import jax.numpy as jnp
from jax import lax
from jax.experimental import pallas as pl
from jax.experimental.pallas import tpu as pltpu
from jax.experimental.pallas import tpu_sc as plsc

_H = 24
_W = 24
_D2 = 384
_D = 768
_P = _H * _W


def _add_body(row_ref, col_ref, x_ref, o_ref):
    xb = x_ref[...]
    o_ref[:, :, :, : _D2] = xb[:, :, :, : _D2] + row_ref[...][None, :, None, :]
    o_ref[:, :, :, _D2:] = xb[:, :, :, _D2:] + col_ref[...][None, None, :, :]


def kernel(x, h, w, row_embed, col_embed):
    B, P, D = x.shape
    x4 = x.reshape(B, _H, _W, D)
    out = pl.pallas_call(
        _add_body,
        grid=(B // 8,),
        in_specs=[
            pl.BlockSpec((_H, _D2), lambda b: (0, 0)),
            pl.BlockSpec((_W, _D2), lambda b: (0, 0)),
            pl.BlockSpec((8, _H, _W, D), lambda b: (b, 0, 0, 0)),
        ],
        out_specs=pl.BlockSpec((8, _H, _W, D), lambda b: (b, 0, 0, 0)),
        out_shape=jax.ShapeDtypeStruct((B, _H, _W, D), jnp.float32),
    )(row_embed, col_embed, x4)
    return out.reshape(B, P, D)
